# baseline (speedup 1.0000x reference)
import functools

import jax
import jax.numpy as jnp
from jax import lax
from jax.experimental import pallas as pl
from jax.experimental.pallas import tpu as pltpu
from jax.experimental.pallas import tpu_sc as plsc

LANE = 128
VOCAB = 1000000
D = 64
K1_BLK = 16384
K1_GRID = 31
TL_ROWS = K1_GRID * K1_BLK
TOP_BLK = VOCAB // K1_BLK


@functools.cache
def _build_k1():
    def body(a_ref, b_ref, o_ref):
        o_ref[:, 0:D] = a_ref[...].T
        o_ref[:, D : 2 * D] = b_ref[...].T

    return pl.pallas_call(
        body,
        grid=(K1_GRID,),
        in_specs=[
            pl.BlockSpec((D, K1_BLK), lambda i: (0, i)),
            pl.BlockSpec((D, K1_BLK), lambda i: (0, TOP_BLK - i)),
        ],
        out_specs=pl.BlockSpec((K1_BLK, 2 * D), lambda i: (i, 0)),
        out_shape=jax.ShapeDtypeStruct((TL_ROWS, 2 * D), jnp.float32),
    )


@functools.cache
def _build_k2(S1, NB, NC, NS):
    NW = NC * NS
    assert NB == NW
    mesh = plsc.VectorSubcoreMesh(core_axis_name="c", subcore_axis_name="s")

    @functools.partial(
        pl.kernel,
        out_type=jax.ShapeDtypeStruct((S1, D // 8, NB, 8 * LANE), jnp.float32),
        mesh=mesh,
        scratch_types=[
            pltpu.VMEM((S1 // 8, 8, LANE), jnp.int32),
            pltpu.VMEM((4, LANE, D), jnp.float32),
            pltpu.VMEM((2, D * LANE), jnp.float32),
            [pltpu.SemaphoreType.DMA] * 4,
            [pltpu.SemaphoreType.DMA] * 2,
        ],
        compiler_params=pltpu.CompilerParams(
            use_tc_tiling_on_sc=False, needs_layout_passes=False
        ),
    )
    def k2(tl_hbm, idx_hbm, out_hbm, idxv, G, O, gsems, wsems):
        w = lax.axis_index("s") * NC + lax.axis_index("c")
        pltpu.sync_copy(idx_hbm.at[:, w], idxv)

        def issue_g(j, b):
            pltpu.async_copy(
                tl_hbm.at[idxv.at[j // 8, j % 8]], G.at[b], gsems[b]
            )

        def wait_g(b):
            pltpu.make_async_copy(
                tl_hbm.at[idxv.at[0, 0]], G.at[b], gsems[b]
            ).wait()

        def issue_w(j, b):
            def one(g, carry):
                pltpu.async_copy(
                    O.at[b, pl.ds(8 * LANE * g, 8 * LANE)],
                    out_hbm.at[j, g, w],
                    wsems[b],
                )
                return carry

            lax.fori_loop(0, D // 8, one, 0)

        def wait_w(b):
            for _ in range(D // 8):
                pltpu.make_async_copy(
                    O.at[b, pl.ds(0, 8 * LANE)], out_hbm.at[0, 0, 0], wsems[b]
                ).wait()

        iota16 = lax.iota(jnp.int32, 16)
        sbases = [iota16 + 16 * m for m in range(8)]

        def transpose_block(b4, b2):
            src = G.at[b4]
            dst = O.at[b2]

            def body(c, cv):
                vs = []
                for u in range(4):
                    wrap = (cv + (iota16 + u)) & (D - 1)
                    vs.append(
                        (wrap * LANE,
                         [plsc.load_gather(src, [sbases[m], wrap])
                          for m in range(8)])
                    )
                for wshift, loads in vs:
                    for m in range(8):
                        plsc.store_scatter(dst, [wshift + sbases[m]], loads[m])
                return cv + 4

            lax.fori_loop(0, D // 4, body, jnp.zeros((16,), jnp.int32))

        for b in range(4):
            issue_g(b, b)
        for j in range(4):
            b4, b2 = j, j % 2
            wait_g(b4)
            if j >= 2:
                wait_w(b2)
            transpose_block(b4, b2)
            issue_w(j, b2)
            issue_g(j + 4, b4)

        def round_body(i, carry):
            for b4 in range(4):
                j = 4 * i + b4
                b2 = b4 % 2
                wait_g(b4)
                wait_w(b2)
                transpose_block(b4, b2)
                issue_w(j, b2)
                issue_g(jnp.minimum(j + 4, S1 - 1), b4)
            return carry

        lax.fori_loop(1, S1 // 4, round_body, 0)

        for b in range(4):
            wait_g(b)
        for b2 in range(2):
            wait_w(b2)

    return k2


def kernel(indices, table):
    S0, S1 = indices.shape
    assert table.shape == (VOCAB, D) and S0 % LANE == 0 and S1 % 8 == 0
    info = plsc.get_sparse_core_info()
    NC, NS = info.num_cores, info.num_subcores
    NW = NC * NS
    NB = S0 // LANE
    assert NB == NW

    table_t = table.T
    tl = _build_k1()(table_t, table_t)
    tl_rows = tl.reshape(2 * TL_ROWS, D)

    idxn = (
        indices.astype(jnp.int32)
        .T.reshape(S1 // 8, 8, NB, LANE)
        .transpose(0, 2, 1, 3)
    )
    low_lim = (TOP_BLK - K1_GRID + 1) * K1_BLK
    top = TOP_BLK * K1_BLK
    hi = 2 * (top - (idxn & ~(K1_BLK - 1)) + (idxn & (K1_BLK - 1))) + 1
    idxn = jnp.where(idxn < low_lim, 2 * idxn, hi)
    p4 = _build_k2(S1, NB, NC, NS)(tl_rows, idxn)
    p5 = p4.reshape(S1, D // 8, NB, 8, LANE)
    return p5.transpose(2, 4, 0, 1, 3).reshape(S0, S1, D)


# speedup vs baseline: 1.1016x; 1.1016x over previous
import functools

import jax
import jax.numpy as jnp
from jax import lax
from jax.experimental import pallas as pl
from jax.experimental.pallas import tpu as pltpu
from jax.experimental.pallas import tpu_sc as plsc

LANE = 128
VOCAB = 1000000
D = 64
K1_BLK = 16384
K1_GRID = 31
TL_ROWS = K1_GRID * K1_BLK
TOP_BLK = VOCAB // K1_BLK


@functools.cache
def _build_k1():
    def body(a_ref, b_ref, o_ref):
        o_ref[:, 0:D] = a_ref[...].T
        o_ref[:, D : 2 * D] = b_ref[...].T

    return pl.pallas_call(
        body,
        grid=(K1_GRID,),
        in_specs=[
            pl.BlockSpec((D, K1_BLK), lambda i: (0, i)),
            pl.BlockSpec((D, K1_BLK), lambda i: (0, TOP_BLK - i)),
        ],
        out_specs=pl.BlockSpec((K1_BLK, 2 * D), lambda i: (i, 0)),
        out_shape=jax.ShapeDtypeStruct((TL_ROWS, 2 * D), jnp.float32),
    )


@functools.cache
def _build_k2(S1, NB, NC, NS):
    NW = NC * NS
    assert NB == NW
    mesh = plsc.VectorSubcoreMesh(core_axis_name="c", subcore_axis_name="s")

    @functools.partial(
        pl.kernel,
        out_type=jax.ShapeDtypeStruct((S1, D // 8, NB, 8 * LANE), jnp.float32),
        mesh=mesh,
        scratch_types=[
            pltpu.VMEM((S1 // 8, 8, LANE), jnp.int32),
            pltpu.VMEM((4, LANE, D), jnp.float32),
            pltpu.VMEM((2, D * LANE), jnp.float32),
            [pltpu.SemaphoreType.DMA] * 4,
            [pltpu.SemaphoreType.DMA] * 2,
        ],
        compiler_params=pltpu.CompilerParams(
            use_tc_tiling_on_sc=False, needs_layout_passes=False
        ),
    )
    def k2(tl_hbm, idx_hbm, out_hbm, idxv, G, O, gsems, wsems):
        w = lax.axis_index("s") * NC + lax.axis_index("c")
        pltpu.sync_copy(idx_hbm.at[:, w], idxv)

        def issue_g(j, b):
            pltpu.async_copy(
                tl_hbm.at[idxv.at[j // 8, j % 8]], G.at[b], gsems[b]
            )

        def wait_g(b):
            pltpu.make_async_copy(
                tl_hbm.at[idxv.at[0, 0]], G.at[b], gsems[b]
            ).wait()

        def issue_w(j, b):
            def one(g, carry):
                pltpu.async_copy(
                    O.at[b, pl.ds(8 * LANE * g, 8 * LANE)],
                    out_hbm.at[j, g, w],
                    wsems[b],
                )
                return carry

            lax.fori_loop(0, D // 8, one, 0)

        def wait_w(b):
            for _ in range(D // 8):
                pltpu.make_async_copy(
                    O.at[b, pl.ds(0, 8 * LANE)], out_hbm.at[0, 0, 0], wsems[b]
                ).wait()

        iota16 = lax.iota(jnp.int32, 16)
        sbases = [iota16 + 16 * m for m in range(8)]

        def transpose_block(b4, b2):
            src = G.at[b4]
            dst = O.at[b2]

            def body(c, cv):
                vs = []
                for u in range(2):
                    wrap = (cv + (iota16 + u)) & (D - 1)
                    vs.append(
                        (wrap * LANE,
                         [plsc.load_gather(src, [sbases[m], wrap])
                          for m in range(8)])
                    )
                for wshift, loads in vs:
                    for m in range(8):
                        plsc.store_scatter(dst, [wshift + sbases[m]], loads[m])
                return cv + 2

            lax.fori_loop(0, D // 2, body, jnp.zeros((16,), jnp.int32))

        for b in range(4):
            issue_g(b, b)
        for j in range(4):
            b4, b2 = j, j % 2
            wait_g(b4)
            if j >= 2:
                wait_w(b2)
            transpose_block(b4, b2)
            issue_w(j, b2)
            issue_g(j + 4, b4)

        def round_body(i, carry):
            for b4 in range(4):
                j = 4 * i + b4
                b2 = b4 % 2
                wait_g(b4)
                wait_w(b2)
                transpose_block(b4, b2)
                issue_w(j, b2)
                issue_g(jnp.minimum(j + 4, S1 - 1), b4)
            return carry

        lax.fori_loop(1, S1 // 4, round_body, 0)

        for b in range(4):
            wait_g(b)
        for b2 in range(2):
            wait_w(b2)

    return k2


def kernel(indices, table):
    S0, S1 = indices.shape
    assert table.shape == (VOCAB, D) and S0 % LANE == 0 and S1 % 8 == 0
    info = plsc.get_sparse_core_info()
    NC, NS = info.num_cores, info.num_subcores
    NW = NC * NS
    NB = S0 // LANE
    assert NB == NW

    table_t = table.T
    tl = _build_k1()(table_t, table_t)
    tl_rows = tl.reshape(2 * TL_ROWS, D)

    idxn = (
        indices.astype(jnp.int32)
        .T.reshape(S1 // 8, 8, NB, LANE)
        .transpose(0, 2, 1, 3)
    )
    low_lim = (TOP_BLK - K1_GRID + 1) * K1_BLK
    top = TOP_BLK * K1_BLK
    hi = 2 * (top - (idxn & ~(K1_BLK - 1)) + (idxn & (K1_BLK - 1))) + 1
    idxn = jnp.where(idxn < low_lim, 2 * idxn, hi)
    p4 = _build_k2(S1, NB, NC, NS)(tl_rows, idxn)
    p5 = p4.reshape(S1, D // 8, NB, 8, LANE)
    return p5.transpose(2, 4, 0, 1, 3).reshape(S0, S1, D)
